# baseline (speedup 1.0000x reference)
import functools
import jax
import jax.numpy as jnp
from jax import lax
from jax.experimental import pallas as pl
from jax.experimental.pallas import tpu as pltpu
from jax.experimental.pallas import tpu_sc as plsc

MARGIN_C = 0.2
NC, NS, L = 2, 16, 16
NW = NC * NS
B, P1, P2, D = 64, 20, 20, 128
P2P = 32
BPW = B // NW
NCH = D // L
JB = 4


def _sqrt16(x):
    x = jnp.maximum(x, 1e-12)
    i = plsc.bitcast(x, jnp.int32)
    i = 0x5F3759DF - (i >> 1)
    y = plsc.bitcast(i, jnp.float32)
    for _ in range(3):
        y = y * (1.5 - 0.5 * x * y * y)
    return x * y


def _sc_body(e1c, e1s, e2c, e2s, g, n2, out,
             a_c, a_s, b_c, b_s, gbuf, d2c, d2s, n2buf, stage):
    wid = lax.axis_index("s") * NC + lax.axis_index("c")
    lane = lax.iota(jnp.int32, L)

    pltpu.sync_copy(n2, n2buf)

    tot_acc = jnp.zeros((L,), jnp.float32)
    cnt_acc = jnp.zeros((L,), jnp.float32)
    for k in range(BPW):
        b = wid * BPW + k
        pltpu.sync_copy(e1c.at[b], a_c)
        pltpu.sync_copy(e1s.at[b], a_s)
        pltpu.sync_copy(e2c.at[b], b_c)
        pltpu.sync_copy(e2s.at[b], b_s)
        pltpu.sync_copy(g.at[b], gbuf)

        for aref, bref, dref in ((a_c, b_c, d2c), (a_s, b_s, d2s)):
            for jb in range(P2 // JB):
                brows = [[bref[JB * jb + r, pl.ds(L * c, L)]
                          for c in range(NCH)] for r in range(JB)]

                def irow(i, _, brows=brows, aref=aref, dref=dref, jb=jb):
                    arow = [aref[i, pl.ds(L * c, L)] for c in range(NCH)]
                    pv = jnp.zeros((L,), jnp.float32)
                    for r in range(JB):
                        acc = jnp.zeros((L,), jnp.float32)
                        for c in range(NCH):
                            dd = arow[c] - brows[r][c]
                            acc = acc + dd * dd
                        tgt_lane = (JB * jb + r) % L
                        pv = jnp.where(lane == tgt_lane, jnp.sum(acc), pv)
                    half = (JB * jb) // L
                    if JB * jb % L == 0:
                        dref[i, pl.ds(L * half, L)] = pv
                    else:
                        dref[i, pl.ds(L * half, L)] = \
                            dref[i, pl.ds(L * half, L)] + pv
                    return 0

                lax.fori_loop(0, P1, irow, 0)

        n2s = plsc.load_gather(n2buf, [jnp.full((L,), b, jnp.int32)])

        def mrow(i, carry):
            tot, cnt = carry
            halves = []
            for h in range(2):
                v = d2c[i, pl.ds(L * h, L)]
                s = d2s[i, pl.ds(L * h, L)]
                dc = _sqrt16(v)
                dsv = _sqrt16(s)
                hs = jnp.minimum(dsv * 0.5, 1.0)
                w = 1.0 - 2.0 * hs * hs
                dist = dc + w * (dsv - dc)
                col_ok = (lane + L * h) < n2s
                dm = jnp.where(col_ok, dist, 100.0)
                gv = gbuf[i, pl.ds(L * h, L)] > 0.0
                halves.append((dm, gv))
            mm0 = jnp.where(halves[0][1], 1e30, halves[0][0])
            mm1 = jnp.where(halves[1][1], 1e30, halves[1][0])
            m = jnp.min(jnp.minimum(mm0, mm1))
            for dm, gv in halves:
                t = dm - m + MARGIN_C
                tot = tot + jnp.where(gv, jnp.maximum(t, 0.0), 0.0)
                cnt = cnt + jnp.where(gv & (t > 0.0), 1.0, 0.0)
            return tot, cnt

        tot_acc, cnt_acc = lax.fori_loop(0, P1, mrow, (tot_acc, cnt_acc))

    tt = jnp.sum(tot_acc)
    cc = jnp.sum(cnt_acc)
    stage[...] = jnp.where(lane == 0, tt, jnp.where(lane == 1, cc, 0.0))
    pltpu.sync_copy(stage, out.at[wid])


@jax.jit
def _run(e1c, e1s, e2c, e2s, gf, n2, lw):
    mesh = plsc.VectorSubcoreMesh(core_axis_name="c", subcore_axis_name="s",
                                  num_cores=NC, num_subcores=NS)
    partials = pl.kernel(
        _sc_body,
        out_type=jax.ShapeDtypeStruct((NW, L), jnp.float32),
        mesh=mesh,
        compiler_params=pltpu.CompilerParams(needs_layout_passes=False),
        scratch_types=[
            pltpu.VMEM((P1, D), jnp.float32),
            pltpu.VMEM((P1, D), jnp.float32),
            pltpu.VMEM((P2, D), jnp.float32),
            pltpu.VMEM((P2, D), jnp.float32),
            pltpu.VMEM((P1, P2P), jnp.float32),
            pltpu.VMEM((P1, P2P), jnp.float32),
            pltpu.VMEM((P1, P2P), jnp.float32),
            pltpu.VMEM((B,), jnp.int32),
            pltpu.VMEM((L,), jnp.float32),
        ],
    )(e1c, e1s, e2c, e2s, gf, n2)
    tot = jnp.sum(partials[:, 0])
    cnt = jnp.sum(partials[:, 1])
    mean = jnp.where(cnt > 0.0, tot / jnp.maximum(cnt, 1.0), MARGIN_C)
    return lw * mean


def kernel(embeddings1_c, embeddings1_s, embeddings2_c, embeddings2_s,
           gt_corr_ms, numPlanes1, numPlanes2, loss_weight):
    gf = jnp.pad(gt_corr_ms.astype(jnp.float32),
                 ((0, 0), (0, 0), (0, P2P - P2)))
    n2 = numPlanes2.reshape(B).astype(jnp.int32)
    lw = jnp.asarray(loss_weight, jnp.float32)
    return _run(embeddings1_c, embeddings1_s, embeddings2_c, embeddings2_s,
                gf, n2, lw)
